# Initial kernel scaffold; baseline (speedup 1.0000x reference)
import jax
import jax.numpy as jnp
from jax.experimental import pallas as pl


def kernel(text, table, W, b):
    raise NotImplementedError("write your pallas kernel here")


# speedup vs baseline: 7.7847x; 7.7847x over previous
import functools

import jax
import jax.numpy as jnp
from jax import lax
from jax.experimental import pallas as pl
from jax.experimental.pallas import tpu as pltpu
from jax.experimental.pallas import tpu_sc as plsc

VOCAB = 1000000
EMBED_DIM = 16
NUM_CLASS = 2
BATCH = 16384
HIST = 200

NC = 2
NS = 16
NW = NC * NS
B_PER_W = BATCH // NW
GROUP = 16
N_GROUPS = B_PER_W // GROUP
IDX_PER_GROUP = GROUP * HIST
GATHER_CHUNK = 128
N_GATHERS = IDX_PER_GROUP // GATHER_CHUNK


def _sc_body(text_hbm, table_hbm, wb_hbm, out_hbm, idx_v, rows_v, pooled_v,
             ocol_v, wb_v, sem):
    wid = lax.axis_index("s") * NC + lax.axis_index("c")

    pltpu.sync_copy(wb_hbm, wb_v)

    def group_body(g, carry):
        base = wid * B_PER_W + g * GROUP
        pltpu.sync_copy(text_hbm.at[pl.ds(base * HIST, IDX_PER_GROUP)],
                        idx_v)
        copies = []
        for j in range(N_GATHERS):
            sl = pl.ds(j * GATHER_CHUNK, GATHER_CHUNK)
            copies.append(
                pltpu.async_copy(table_hbm.at[idx_v.at[sl]],
                                 rows_v.at[sl], sem))
        for c in copies:
            c.wait()

        for r in range(GROUP):
            def acc_body(l, acc):
                o = r * HIST + l * 8
                for k in range(8):
                    acc = acc + rows_v[o + k, :]
                return acc

            pooled_v[r, :] = lax.fori_loop(0, HIST // 8, acc_body,
                                           jnp.zeros((16,), jnp.float32))

        lanes = jnp.arange(16, dtype=jnp.int32)
        w0row = wb_v[0, :]
        w1row = wb_v[1, :]
        out0 = jnp.full((16,), wb_v[2, :][0], jnp.float32)
        out1 = jnp.full((16,), wb_v[3, :][0], jnp.float32)
        for d in range(EMBED_DIM):
            col = plsc.load_gather(
                pooled_v, [lanes, jnp.full((16,), d, jnp.int32)])
            out0 = out0 + col * w0row[d]
            out1 = out1 + col * w1row[d]
        ocol_v[0, :] = out0
        ocol_v[1, :] = out1
        pltpu.sync_copy(ocol_v.at[0], out_hbm.at[0, pl.ds(base, GROUP)])
        pltpu.sync_copy(ocol_v.at[1], out_hbm.at[1, pl.ds(base, GROUP)])
        return carry

    lax.fori_loop(0, N_GROUPS, group_body, 0)


@jax.jit
def _classify(text_flat, table, wb):
    mesh = plsc.VectorSubcoreMesh(core_axis_name="c", subcore_axis_name="s")
    kern = functools.partial(
        pl.kernel,
        mesh=mesh,
        out_type=jax.ShapeDtypeStruct((NUM_CLASS, BATCH), jnp.float32),
        scratch_types=[
            pltpu.VMEM((IDX_PER_GROUP,), jnp.int32),
            pltpu.VMEM((IDX_PER_GROUP, EMBED_DIM), jnp.float32),
            pltpu.VMEM((GROUP, EMBED_DIM), jnp.float32),
            pltpu.VMEM((NUM_CLASS, GROUP), jnp.float32),
            pltpu.VMEM((4, 16), jnp.float32),
            pltpu.SemaphoreType.DMA,
        ],
        compiler_params=pltpu.CompilerParams(needs_layout_passes=False,
                                             use_tc_tiling_on_sc=False),
    )(_sc_body)
    return kern(text_flat, table, wb)


def kernel(text, table, W, b):
    wt = jnp.transpose(W) * (1.0 / HIST)
    bb = jnp.broadcast_to(b[:, None], (NUM_CLASS, 16))
    wb = jnp.concatenate([wt, bb], axis=0).astype(jnp.float32)
    out = _classify(text.reshape(-1), table, wb)
    return jnp.transpose(out)


# speedup vs baseline: 9.7590x; 1.2536x over previous
import functools

import jax
import jax.numpy as jnp
from jax import lax
from jax.experimental import pallas as pl
from jax.experimental.pallas import tpu as pltpu
from jax.experimental.pallas import tpu_sc as plsc

VOCAB = 1000000
EMBED_DIM = 16
NUM_CLASS = 2
BATCH = 16384
HIST = 200

NC = 2
NS = 16
NW = NC * NS
B_PER_W = BATCH // NW
GROUP = 16
N_GROUPS = B_PER_W // GROUP
N_PAIRS = N_GROUPS // 2
IDX_PER_GROUP = GROUP * HIST
GATHER_CHUNK = 128
N_GATHERS = IDX_PER_GROUP // GATHER_CHUNK


def _sc_body(text_hbm, table_hbm, wb_hbm, out_hbm, idx0_v, idx1_v, rows0_v,
             rows1_v, pooled_v, ocol_v, wb_v, sem0, sem1):
    wid = lax.axis_index("s") * NC + lax.axis_index("c")
    row0 = wid * B_PER_W
    idx_bufs = (idx0_v, idx1_v)
    rows_bufs = (rows0_v, rows1_v)
    sems = (sem0, sem1)

    pltpu.sync_copy(wb_hbm, wb_v)

    def stage_and_fire(g, b):
        base = row0 + g * GROUP
        pltpu.sync_copy(text_hbm.at[pl.ds(base * HIST, IDX_PER_GROUP)],
                        idx_bufs[b])
        for j in range(N_GATHERS):
            sl = pl.ds(j * GATHER_CHUNK, GATHER_CHUNK)
            pltpu.async_copy(table_hbm.at[idx_bufs[b].at[sl]],
                             rows_bufs[b].at[sl], sems[b])

    def drain(b):
        for j in range(N_GATHERS):
            sl = pl.ds(j * GATHER_CHUNK, GATHER_CHUNK)
            pltpu.make_async_copy(table_hbm.at[idx_bufs[b].at[sl]],
                                  rows_bufs[b].at[sl], sems[b]).wait()

    def pool_and_classify(g, b):
        rows_v = rows_bufs[b]
        for r in range(GROUP):
            def acc_body(l, accs):
                o = r * HIST + l * 8
                a0, a1, a2, a3 = accs
                a0 = a0 + rows_v[o + 0, :] + rows_v[o + 4, :]
                a1 = a1 + rows_v[o + 1, :] + rows_v[o + 5, :]
                a2 = a2 + rows_v[o + 2, :] + rows_v[o + 6, :]
                a3 = a3 + rows_v[o + 3, :] + rows_v[o + 7, :]
                return a0, a1, a2, a3

            z = jnp.zeros((16,), jnp.float32)
            a0, a1, a2, a3 = lax.fori_loop(0, HIST // 8, acc_body,
                                           (z, z, z, z))
            pooled_v[r, :] = (a0 + a1) + (a2 + a3)

        lanes = jnp.arange(16, dtype=jnp.int32)
        w0row = wb_v[0, :]
        w1row = wb_v[1, :]
        out0 = jnp.full((16,), wb_v[2, :][0], jnp.float32)
        out1 = jnp.full((16,), wb_v[3, :][0], jnp.float32)
        for d in range(EMBED_DIM):
            col = plsc.load_gather(
                pooled_v, [lanes, jnp.full((16,), d, jnp.int32)])
            out0 = out0 + col * w0row[d]
            out1 = out1 + col * w1row[d]
        ocol_v[0, :] = out0
        ocol_v[1, :] = out1
        base = row0 + g * GROUP
        pltpu.sync_copy(ocol_v.at[0], out_hbm.at[0, pl.ds(base, GROUP)])
        pltpu.sync_copy(ocol_v.at[1], out_hbm.at[1, pl.ds(base, GROUP)])

    stage_and_fire(0, 0)

    def pair_body(p, carry):
        g0 = p * 2
        stage_and_fire(g0 + 1, 1)
        drain(0)
        pool_and_classify(g0, 0)

        @pl.when(p < N_PAIRS - 1)
        def _():
            stage_and_fire(g0 + 2, 0)

        drain(1)
        pool_and_classify(g0 + 1, 1)
        return carry

    lax.fori_loop(0, N_PAIRS, pair_body, 0)


@jax.jit
def _classify(text_flat, table, wb):
    mesh = plsc.VectorSubcoreMesh(core_axis_name="c", subcore_axis_name="s")
    kern = functools.partial(
        pl.kernel,
        mesh=mesh,
        out_type=jax.ShapeDtypeStruct((NUM_CLASS, BATCH), jnp.float32),
        scratch_types=[
            pltpu.VMEM((IDX_PER_GROUP,), jnp.int32),
            pltpu.VMEM((IDX_PER_GROUP,), jnp.int32),
            pltpu.VMEM((IDX_PER_GROUP, EMBED_DIM), jnp.float32),
            pltpu.VMEM((IDX_PER_GROUP, EMBED_DIM), jnp.float32),
            pltpu.VMEM((GROUP, EMBED_DIM), jnp.float32),
            pltpu.VMEM((NUM_CLASS, GROUP), jnp.float32),
            pltpu.VMEM((4, 16), jnp.float32),
            pltpu.SemaphoreType.DMA,
            pltpu.SemaphoreType.DMA,
        ],
        compiler_params=pltpu.CompilerParams(needs_layout_passes=False,
                                             use_tc_tiling_on_sc=False),
    )(_sc_body)
    return kern(text_flat, table, wb)


def kernel(text, table, W, b):
    wt = jnp.transpose(W) * (1.0 / HIST)
    bb = jnp.broadcast_to(b[:, None], (NUM_CLASS, 16))
    wb = jnp.concatenate([wt, bb], axis=0).astype(jnp.float32)
    out = _classify(text.reshape(-1), table, wb)
    return jnp.transpose(out)


# speedup vs baseline: 10.3214x; 1.0576x over previous
import functools

import jax
import jax.numpy as jnp
from jax import lax
from jax.experimental import pallas as pl
from jax.experimental.pallas import tpu as pltpu
from jax.experimental.pallas import tpu_sc as plsc

VOCAB = 1000000
EMBED_DIM = 16
NUM_CLASS = 2
BATCH = 16384
HIST = 200

NC = 2
NS = 16
NW = NC * NS

SUP = 1024
NSUP = VOCAB // SUP + 1
TAIL_START = (VOCAB // SUP) * SUP
L_PER_W = 31
XROWS = NSUP * 128

B_PER_W = BATCH // NW
GROUP = 16
N_GROUPS = B_PER_W // GROUP
N_PAIRS = N_GROUPS // 2
IDX_PER_GROUP = GROUP * HIST
GATHER_CHUNK = 128
N_GATHERS = IDX_PER_GROUP // GATHER_CHUNK


def _a_body(tt_hbm, tail_hbm, x_hbm, blk0_v, blk1_v, out_v, semi0, semi1):
    wid = lax.axis_index("s") * NC + lax.axis_index("c")
    blks = (blk0_v, blk1_v)
    sems = (semi0, semi1)
    lanes = jnp.arange(16, dtype=jnp.int32)
    z16 = jnp.zeros((16,), jnp.int32)

    def su_of(l):
        return wid + 32 * l

    def fire(l, b):
        su = su_of(l)

        @pl.when(su < NSUP - 1)
        def _():
            pltpu.async_copy(tt_hbm.at[:, pl.ds(su * SUP, SUP)], blks[b],
                             sems[b])

        @pl.when(su == NSUP - 1)
        def _():
            pltpu.async_copy(tail_hbm, blks[b], sems[b])

    def wait_in(b):
        pltpu.make_async_copy(tt_hbm.at[:, pl.ds(0, SUP)], blks[b],
                              sems[b]).wait()

    def compute(l, b):
        blk_v = blks[b]

        def row_body(r, carry):
            for k in range(8):
                col = plsc.load_gather(blk_v, [lanes, z16 + (8 * r + k)])
                out_v[r, pl.ds(k * 16, 16)] = col
            return carry

        lax.fori_loop(0, 128, row_body, 0)
        su = su_of(l)
        pltpu.sync_copy(out_v, x_hbm.at[pl.ds(su * 128, 128), :])

    def valid(l):
        return su_of(l) < NSUP

    fire(0, 0)

    def pair_body(p, carry):
        l0 = 2 * p
        l1 = l0 + 1

        @pl.when(valid(l1))
        def _():
            fire(l1, 1)

        @pl.when(valid(l0))
        def _():
            wait_in(0)
            compute(l0, 0)

        @pl.when(valid(l0 + 2))
        def _():
            fire(l0 + 2, 0)

        @pl.when(valid(l1))
        def _():
            wait_in(1)
            compute(l1, 1)

        return carry

    lax.fori_loop(0, (L_PER_W + 1) // 2, pair_body, 0)


@jax.jit
def _detranspose(tt, tail):
    mesh = plsc.VectorSubcoreMesh(core_axis_name="c", subcore_axis_name="s")
    kern = functools.partial(
        pl.kernel,
        mesh=mesh,
        out_type=jax.ShapeDtypeStruct((XROWS, 128), jnp.float32),
        scratch_types=[
            pltpu.VMEM((16, SUP), jnp.float32),
            pltpu.VMEM((16, SUP), jnp.float32),
            pltpu.VMEM((128, 128), jnp.float32),
            pltpu.SemaphoreType.DMA,
            pltpu.SemaphoreType.DMA,
        ],
        compiler_params=pltpu.CompilerParams(needs_layout_passes=False,
                                             use_tc_tiling_on_sc=True),
    )(_a_body)
    return kern(tt, tail)


def _sc_body(text_hbm, table_hbm, wb_hbm, out_hbm, idx0_v, idx1_v, rows0_v,
             rows1_v, pooled_v, ocol_v, wb_v, sem0, sem1):
    wid = lax.axis_index("s") * NC + lax.axis_index("c")
    row0 = wid * B_PER_W
    idx_bufs = (idx0_v, idx1_v)
    rows_bufs = (rows0_v, rows1_v)
    sems = (sem0, sem1)

    pltpu.sync_copy(wb_hbm, wb_v)

    def stage_and_fire(g, b):
        base = row0 + g * GROUP
        pltpu.sync_copy(text_hbm.at[pl.ds(base * HIST, IDX_PER_GROUP)],
                        idx_bufs[b])
        for j in range(N_GATHERS):
            sl = pl.ds(j * GATHER_CHUNK, GATHER_CHUNK)
            pltpu.async_copy(table_hbm.at[idx_bufs[b].at[sl]],
                             rows_bufs[b].at[sl], sems[b])

    def drain(b):
        for j in range(N_GATHERS):
            sl = pl.ds(j * GATHER_CHUNK, GATHER_CHUNK)
            pltpu.make_async_copy(table_hbm.at[idx_bufs[b].at[sl]],
                                  rows_bufs[b].at[sl], sems[b]).wait()

    def pool_and_classify(g, b):
        rows_v = rows_bufs[b]
        for r in range(GROUP):
            def acc_body(l, accs):
                o = r * HIST + l * 8
                a0, a1, a2, a3 = accs
                a0 = a0 + rows_v[o + 0, :] + rows_v[o + 4, :]
                a1 = a1 + rows_v[o + 1, :] + rows_v[o + 5, :]
                a2 = a2 + rows_v[o + 2, :] + rows_v[o + 6, :]
                a3 = a3 + rows_v[o + 3, :] + rows_v[o + 7, :]
                return a0, a1, a2, a3

            z = jnp.zeros((16,), jnp.float32)
            a0, a1, a2, a3 = lax.fori_loop(0, HIST // 8, acc_body,
                                           (z, z, z, z))
            pooled_v[r, :] = (a0 + a1) + (a2 + a3)

        lanes = jnp.arange(16, dtype=jnp.int32)
        w0row = wb_v[0, :]
        w1row = wb_v[1, :]
        out0 = jnp.full((16,), wb_v[2, :][0], jnp.float32)
        out1 = jnp.full((16,), wb_v[3, :][0], jnp.float32)
        for d in range(EMBED_DIM):
            col = plsc.load_gather(
                pooled_v, [lanes, jnp.full((16,), d, jnp.int32)])
            out0 = out0 + col * w0row[d]
            out1 = out1 + col * w1row[d]
        ocol_v[0, :] = out0
        ocol_v[1, :] = out1
        base = row0 + g * GROUP
        pltpu.sync_copy(ocol_v.at[0], out_hbm.at[0, pl.ds(base, GROUP)])
        pltpu.sync_copy(ocol_v.at[1], out_hbm.at[1, pl.ds(base, GROUP)])

    stage_and_fire(0, 0)

    def pair_body(p, carry):
        g0 = p * 2
        stage_and_fire(g0 + 1, 1)
        drain(0)
        pool_and_classify(g0, 0)

        @pl.when(p < N_PAIRS - 1)
        def _():
            stage_and_fire(g0 + 2, 0)

        drain(1)
        pool_and_classify(g0 + 1, 1)
        return carry

    lax.fori_loop(0, N_PAIRS, pair_body, 0)


@jax.jit
def _classify(text_flat, table_lin, wb):
    mesh = plsc.VectorSubcoreMesh(core_axis_name="c", subcore_axis_name="s")
    kern = functools.partial(
        pl.kernel,
        mesh=mesh,
        out_type=jax.ShapeDtypeStruct((NUM_CLASS, BATCH), jnp.float32),
        scratch_types=[
            pltpu.VMEM((IDX_PER_GROUP,), jnp.int32),
            pltpu.VMEM((IDX_PER_GROUP,), jnp.int32),
            pltpu.VMEM((IDX_PER_GROUP, EMBED_DIM), jnp.float32),
            pltpu.VMEM((IDX_PER_GROUP, EMBED_DIM), jnp.float32),
            pltpu.VMEM((GROUP, EMBED_DIM), jnp.float32),
            pltpu.VMEM((NUM_CLASS, GROUP), jnp.float32),
            pltpu.VMEM((4, 16), jnp.float32),
            pltpu.SemaphoreType.DMA,
            pltpu.SemaphoreType.DMA,
        ],
        compiler_params=pltpu.CompilerParams(needs_layout_passes=False,
                                             use_tc_tiling_on_sc=False),
    )(_sc_body)
    return kern(text_flat, table_lin, wb)


def kernel(text, table, W, b):
    tt = jnp.transpose(table)
    tail = lax.slice(tt, (0, TAIL_START), (EMBED_DIM, VOCAB))
    tail = jnp.pad(tail, ((0, 0), (0, SUP - (VOCAB - TAIL_START))))
    x = _detranspose(tt, tail)
    table_lin = x.reshape(XROWS * 8, EMBED_DIM)
    wt = jnp.transpose(W) * (1.0 / HIST)
    bb = jnp.broadcast_to(b[:, None], (NUM_CLASS, 16))
    wb = jnp.concatenate([wt, bb], axis=0).astype(jnp.float32)
    out = _classify(text.reshape(-1), table_lin, wb)
    return jnp.transpose(out)


# speedup vs baseline: 24.0000x; 2.3253x over previous
import functools

import jax
import jax.numpy as jnp
from jax import lax
from jax.experimental import pallas as pl
from jax.experimental.pallas import tpu as pltpu
from jax.experimental.pallas import tpu_sc as plsc

VOCAB = 1000000
EMBED_DIM = 16
NUM_CLASS = 2
BATCH = 16384
HIST = 200

NC = 2
NS = 16
NW = NC * NS

SUP = 1024
NSUP = VOCAB // SUP + 1
TAIL_START = (VOCAB // SUP) * SUP
L_PER_W = 31
XROWS = NSUP * 128

B_PER_W = BATCH // NW
GROUP = 16
N_GROUPS = B_PER_W // GROUP
N_PAIRS = N_GROUPS // 2
IDX_PER_GROUP = GROUP * HIST
GATHER_CHUNK = 128
N_GATHERS = IDX_PER_GROUP // GATHER_CHUNK


def _a_body(tt_hbm, tail_hbm, x_hbm, blk0_v, blk1_v, out0_v, out1_v,
            semi0, semi1, semo0, semo1):
    wid = lax.axis_index("s") * NC + lax.axis_index("c")
    blks = (blk0_v, blk1_v)
    outs = (out0_v, out1_v)
    semis = (semi0, semi1)
    semos = (semo0, semo1)
    lanes = jnp.arange(16, dtype=jnp.int32)
    row_pat = lanes >> 3
    col_pat = (lanes & 7) * 16

    def su_of(l):
        return wid + 32 * l

    def fire(l, b):
        su = su_of(l)

        @pl.when(su < NSUP - 1)
        def _():
            pltpu.async_copy(tt_hbm.at[:, pl.ds(su * SUP, SUP)], blks[b],
                             semis[b])

        @pl.when(su == NSUP - 1)
        def _():
            pltpu.async_copy(tail_hbm, blks[b], semis[b])

    def wait_in(b):
        pltpu.make_async_copy(tt_hbm.at[:, pl.ds(0, SUP)], blks[b],
                              semis[b]).wait()

    def wait_out(b):
        pltpu.make_async_copy(outs[b], x_hbm.at[pl.ds(0, 128), :],
                              semos[b]).wait()

    def compute(l, b):
        blk_v = blks[b]
        out_v = outs[b]

        @plsc.parallel_loop(0, SUP // 16, unroll=2)
        def chunk_body(c):
            rowv = row_pat + 2 * c
            for d in range(EMBED_DIM):
                v = blk_v[d, pl.ds(c * 16, 16)]
                plsc.store_scatter(out_v, [rowv, col_pat + d], v)

        su = su_of(l)
        pltpu.async_copy(out_v, x_hbm.at[pl.ds(su * 128, 128), :], semos[b])

    def valid(l):
        return su_of(l) < NSUP

    fire(0, 0)

    def pair_body(p, carry):
        l0 = 2 * p
        l1 = l0 + 1

        @pl.when(valid(l1))
        def _():
            fire(l1, 1)

        @pl.when(valid(l0))
        def _():
            wait_in(0)

            @pl.when(p > 0)
            def _():
                wait_out(0)

            compute(l0, 0)

        @pl.when(valid(l0 + 2))
        def _():
            fire(l0 + 2, 0)

        @pl.when(valid(l1))
        def _():
            wait_in(1)

            @pl.when(p > 0)
            def _():
                wait_out(1)

            compute(l1, 1)

        return carry

    lax.fori_loop(0, (L_PER_W + 1) // 2, pair_body, 0)
    wait_out(0)

    @pl.when(valid(1))
    def _():
        wait_out(1)


@jax.jit
def _detranspose(tt, tail):
    mesh = plsc.VectorSubcoreMesh(core_axis_name="c", subcore_axis_name="s")
    kern = functools.partial(
        pl.kernel,
        mesh=mesh,
        out_type=jax.ShapeDtypeStruct((XROWS, 128), jnp.float32),
        scratch_types=[
            pltpu.VMEM((16, SUP), jnp.float32),
            pltpu.VMEM((16, SUP), jnp.float32),
            pltpu.VMEM((128, 128), jnp.float32),
            pltpu.VMEM((128, 128), jnp.float32),
            pltpu.SemaphoreType.DMA,
            pltpu.SemaphoreType.DMA,
            pltpu.SemaphoreType.DMA,
            pltpu.SemaphoreType.DMA,
        ],
        compiler_params=pltpu.CompilerParams(needs_layout_passes=False,
                                             use_tc_tiling_on_sc=True),
    )(_a_body)
    return kern(tt, tail)


def _sc_body(text_hbm, table_hbm, wb_hbm, out_hbm, idx0_v, idx1_v, rows0_v,
             rows1_v, pooled_v, ocol_v, wb_v, sem0, sem1):
    wid = lax.axis_index("s") * NC + lax.axis_index("c")
    row0 = wid * B_PER_W
    idx_bufs = (idx0_v, idx1_v)
    rows_bufs = (rows0_v, rows1_v)
    sems = (sem0, sem1)

    pltpu.sync_copy(wb_hbm, wb_v)

    def stage_and_fire(g, b):
        base = row0 + g * GROUP
        pltpu.sync_copy(text_hbm.at[pl.ds(base * HIST, IDX_PER_GROUP)],
                        idx_bufs[b])
        for j in range(N_GATHERS):
            sl = pl.ds(j * GATHER_CHUNK, GATHER_CHUNK)
            pltpu.async_copy(table_hbm.at[idx_bufs[b].at[sl]],
                             rows_bufs[b].at[sl], sems[b])

    def drain(b):
        for j in range(N_GATHERS):
            sl = pl.ds(j * GATHER_CHUNK, GATHER_CHUNK)
            pltpu.make_async_copy(table_hbm.at[idx_bufs[b].at[sl]],
                                  rows_bufs[b].at[sl], sems[b]).wait()

    def pool_and_classify(g, b):
        rows_v = rows_bufs[b]
        for r in range(GROUP):
            def acc_body(l, accs):
                o = r * HIST + l * 8
                a0, a1, a2, a3 = accs
                a0 = a0 + rows_v[o + 0, :] + rows_v[o + 4, :]
                a1 = a1 + rows_v[o + 1, :] + rows_v[o + 5, :]
                a2 = a2 + rows_v[o + 2, :] + rows_v[o + 6, :]
                a3 = a3 + rows_v[o + 3, :] + rows_v[o + 7, :]
                return a0, a1, a2, a3

            z = jnp.zeros((16,), jnp.float32)
            a0, a1, a2, a3 = lax.fori_loop(0, HIST // 8, acc_body,
                                           (z, z, z, z))
            pooled_v[r, :] = (a0 + a1) + (a2 + a3)

        lanes = jnp.arange(16, dtype=jnp.int32)
        w0row = wb_v[0, :]
        w1row = wb_v[1, :]
        out0 = jnp.full((16,), wb_v[2, :][0], jnp.float32)
        out1 = jnp.full((16,), wb_v[3, :][0], jnp.float32)
        for d in range(EMBED_DIM):
            col = plsc.load_gather(
                pooled_v, [lanes, jnp.full((16,), d, jnp.int32)])
            out0 = out0 + col * w0row[d]
            out1 = out1 + col * w1row[d]
        ocol_v[0, :] = out0
        ocol_v[1, :] = out1
        base = row0 + g * GROUP
        pltpu.sync_copy(ocol_v.at[0], out_hbm.at[0, pl.ds(base, GROUP)])
        pltpu.sync_copy(ocol_v.at[1], out_hbm.at[1, pl.ds(base, GROUP)])

    stage_and_fire(0, 0)

    def pair_body(p, carry):
        g0 = p * 2
        stage_and_fire(g0 + 1, 1)
        drain(0)
        pool_and_classify(g0, 0)

        @pl.when(p < N_PAIRS - 1)
        def _():
            stage_and_fire(g0 + 2, 0)

        drain(1)
        pool_and_classify(g0 + 1, 1)
        return carry

    lax.fori_loop(0, N_PAIRS, pair_body, 0)


@jax.jit
def _classify(text_flat, table_lin, wb):
    mesh = plsc.VectorSubcoreMesh(core_axis_name="c", subcore_axis_name="s")
    kern = functools.partial(
        pl.kernel,
        mesh=mesh,
        out_type=jax.ShapeDtypeStruct((NUM_CLASS, BATCH), jnp.float32),
        scratch_types=[
            pltpu.VMEM((IDX_PER_GROUP,), jnp.int32),
            pltpu.VMEM((IDX_PER_GROUP,), jnp.int32),
            pltpu.VMEM((IDX_PER_GROUP, EMBED_DIM), jnp.float32),
            pltpu.VMEM((IDX_PER_GROUP, EMBED_DIM), jnp.float32),
            pltpu.VMEM((GROUP, EMBED_DIM), jnp.float32),
            pltpu.VMEM((NUM_CLASS, GROUP), jnp.float32),
            pltpu.VMEM((4, 16), jnp.float32),
            pltpu.SemaphoreType.DMA,
            pltpu.SemaphoreType.DMA,
        ],
        compiler_params=pltpu.CompilerParams(needs_layout_passes=False,
                                             use_tc_tiling_on_sc=False),
    )(_sc_body)
    return kern(text_flat, table_lin, wb)


def kernel(text, table, W, b):
    tt = jnp.transpose(table)
    tail = lax.slice(tt, (0, TAIL_START), (EMBED_DIM, VOCAB))
    tail = jnp.pad(tail, ((0, 0), (0, SUP - (VOCAB - TAIL_START))))
    x = _detranspose(tt, tail)
    table_lin = x.reshape(XROWS * 8, EMBED_DIM)
    wt = jnp.transpose(W) * (1.0 / HIST)
    bb = jnp.broadcast_to(b[:, None], (NUM_CLASS, 16))
    wb = jnp.concatenate([wt, bb], axis=0).astype(jnp.float32)
    out = _classify(text.reshape(-1), table_lin, wb)
    return jnp.transpose(out)


# speedup vs baseline: 24.1562x; 1.0065x over previous
import functools

import jax
import jax.numpy as jnp
from jax import lax
from jax.experimental import pallas as pl
from jax.experimental.pallas import tpu as pltpu
from jax.experimental.pallas import tpu_sc as plsc

VOCAB = 1000000
EMBED_DIM = 16
NUM_CLASS = 2
BATCH = 16384
HIST = 200

NC = 2
NS = 16
NW = NC * NS

SUP = 1024
NSUP = VOCAB // SUP + 1
TAIL_START = (VOCAB // SUP) * SUP
L_PER_W = 31
XROWS = NSUP * 128

B_PER_W = BATCH // NW
GROUP = 16
N_GROUPS = B_PER_W // GROUP
N_PAIRS = N_GROUPS // 2
IDX_PER_GROUP = GROUP * HIST
GATHER_CHUNK = 128
N_GATHERS = IDX_PER_GROUP // GATHER_CHUNK


def _a_body(tt_hbm, tail_hbm, x_hbm, blk0_v, blk1_v, out0_v, out1_v,
            semi0, semi1, semo0, semo1):
    wid = lax.axis_index("s") * NC + lax.axis_index("c")
    blks = (blk0_v, blk1_v)
    outs = (out0_v, out1_v)
    semis = (semi0, semi1)
    semos = (semo0, semo1)
    lanes = jnp.arange(16, dtype=jnp.int32)
    row_pat = lanes >> 3
    col_pat = (lanes & 7) * 16

    def su_of(l):
        return wid + 32 * l

    def fire(l, b):
        su = su_of(l)

        @pl.when(su < NSUP - 1)
        def _():
            pltpu.async_copy(tt_hbm.at[:, pl.ds(su * SUP, SUP)], blks[b],
                             semis[b])

        @pl.when(su == NSUP - 1)
        def _():
            pltpu.async_copy(tail_hbm, blks[b], semis[b])

    def wait_in(b):
        pltpu.make_async_copy(tt_hbm.at[:, pl.ds(0, SUP)], blks[b],
                              semis[b]).wait()

    def wait_out(b):
        pltpu.make_async_copy(outs[b], x_hbm.at[pl.ds(0, 128), :],
                              semos[b]).wait()

    def compute(l, b):
        blk_v = blks[b]
        out_v = outs[b]

        @plsc.parallel_loop(0, SUP // 16, unroll=2)
        def chunk_body(c):
            rowv = row_pat + 2 * c
            for d in range(EMBED_DIM):
                v = blk_v[d, pl.ds(c * 16, 16)]
                plsc.store_scatter(out_v, [rowv, col_pat + d], v)

        su = su_of(l)
        pltpu.async_copy(out_v, x_hbm.at[pl.ds(su * 128, 128), :], semos[b])

    def valid(l):
        return su_of(l) < NSUP

    fire(0, 0)

    def pair_body(p, carry):
        l0 = 2 * p
        l1 = l0 + 1

        @pl.when(valid(l1))
        def _():
            fire(l1, 1)

        @pl.when(valid(l0))
        def _():
            wait_in(0)

            @pl.when(p > 0)
            def _():
                wait_out(0)

            compute(l0, 0)

        @pl.when(valid(l0 + 2))
        def _():
            fire(l0 + 2, 0)

        @pl.when(valid(l1))
        def _():
            wait_in(1)

            @pl.when(p > 0)
            def _():
                wait_out(1)

            compute(l1, 1)

        return carry

    lax.fori_loop(0, (L_PER_W + 1) // 2, pair_body, 0)
    wait_out(0)

    @pl.when(valid(1))
    def _():
        wait_out(1)


@jax.jit
def _detranspose(tt, tail):
    mesh = plsc.VectorSubcoreMesh(core_axis_name="c", subcore_axis_name="s")
    kern = functools.partial(
        pl.kernel,
        mesh=mesh,
        out_type=jax.ShapeDtypeStruct((XROWS, 128), jnp.float32),
        scratch_types=[
            pltpu.VMEM((16, SUP), jnp.float32),
            pltpu.VMEM((16, SUP), jnp.float32),
            pltpu.VMEM((128, 128), jnp.float32),
            pltpu.VMEM((128, 128), jnp.float32),
            pltpu.SemaphoreType.DMA,
            pltpu.SemaphoreType.DMA,
            pltpu.SemaphoreType.DMA,
            pltpu.SemaphoreType.DMA,
        ],
        compiler_params=pltpu.CompilerParams(needs_layout_passes=False,
                                             use_tc_tiling_on_sc=True),
    )(_a_body)
    return kern(tt, tail)


def _sc_body(text_hbm, table_hbm, wb_hbm, out_hbm, idx0_v, idx1_v, rows0_v,
             rows1_v, pooled_v, ocol0_v, ocol1_v, wb_v, sem0, sem1, semx0,
             semx1, semo):
    wid = lax.axis_index("s") * NC + lax.axis_index("c")
    row0 = wid * B_PER_W
    idx_bufs = (idx0_v, idx1_v)
    rows_bufs = (rows0_v, rows1_v)
    ocol_bufs = (ocol0_v, ocol1_v)
    sems = (sem0, sem1)
    semxs = (semx0, semx1)

    pltpu.sync_copy(wb_hbm, wb_v)

    def stage_idx(g, b):
        base = row0 + g * GROUP
        pltpu.async_copy(text_hbm.at[pl.ds(base * HIST, IDX_PER_GROUP)],
                         idx_bufs[b], semxs[b])

    def wait_idx(b):
        pltpu.make_async_copy(text_hbm.at[pl.ds(0, IDX_PER_GROUP)],
                              idx_bufs[b], semxs[b]).wait()

    def fire(b):
        for j in range(N_GATHERS):
            sl = pl.ds(j * GATHER_CHUNK, GATHER_CHUNK)
            pltpu.async_copy(table_hbm.at[idx_bufs[b].at[sl]],
                             rows_bufs[b].at[sl], sems[b])

    def drain(b):
        for j in range(N_GATHERS):
            sl = pl.ds(j * GATHER_CHUNK, GATHER_CHUNK)
            pltpu.make_async_copy(table_hbm.at[idx_bufs[b].at[sl]],
                                  rows_bufs[b].at[sl], sems[b]).wait()

    def pool_and_classify(g, b):
        rows_v = rows_bufs[b]
        for r in range(GROUP):
            def acc_body(l, accs):
                o = r * HIST + l * 8
                a0, a1, a2, a3 = accs
                a0 = a0 + rows_v[o + 0, :] + rows_v[o + 4, :]
                a1 = a1 + rows_v[o + 1, :] + rows_v[o + 5, :]
                a2 = a2 + rows_v[o + 2, :] + rows_v[o + 6, :]
                a3 = a3 + rows_v[o + 3, :] + rows_v[o + 7, :]
                return a0, a1, a2, a3

            z = jnp.zeros((16,), jnp.float32)
            a0, a1, a2, a3 = lax.fori_loop(0, HIST // 8, acc_body,
                                           (z, z, z, z))
            pooled_v[r, :] = (a0 + a1) + (a2 + a3)

        lanes = jnp.arange(16, dtype=jnp.int32)
        w0row = wb_v[0, :]
        w1row = wb_v[1, :]
        out0 = jnp.full((16,), wb_v[2, :][0], jnp.float32)
        out1 = jnp.full((16,), wb_v[3, :][0], jnp.float32)
        for d in range(EMBED_DIM):
            col = plsc.load_gather(
                pooled_v, [lanes, jnp.full((16,), d, jnp.int32)])
            out0 = out0 + col * w0row[d]
            out1 = out1 + col * w1row[d]
        ocol_v = ocol_bufs[b]
        ocol_v[0, :] = out0
        ocol_v[1, :] = out1
        base = row0 + g * GROUP
        pltpu.sync_copy(ocol_v.at[0], out_hbm.at[0, pl.ds(base, GROUP)])
        pltpu.sync_copy(ocol_v.at[1], out_hbm.at[1, pl.ds(base, GROUP)])

    stage_idx(0, 0)
    wait_idx(0)
    fire(0)
    stage_idx(1, 1)

    def pair_body(p, carry):
        g0 = p * 2
        wait_idx(1)
        fire(1)
        drain(0)

        @pl.when(p < N_PAIRS - 1)
        def _():
            stage_idx(g0 + 2, 0)

        pool_and_classify(g0, 0)

        @pl.when(p < N_PAIRS - 1)
        def _():
            wait_idx(0)
            fire(0)

        drain(1)

        @pl.when(p < N_PAIRS - 1)
        def _():
            stage_idx(g0 + 3, 1)

        pool_and_classify(g0 + 1, 1)
        return carry

    lax.fori_loop(0, N_PAIRS, pair_body, 0)


@jax.jit
def _classify(text_flat, table_lin, wb):
    mesh = plsc.VectorSubcoreMesh(core_axis_name="c", subcore_axis_name="s")
    kern = functools.partial(
        pl.kernel,
        mesh=mesh,
        out_type=jax.ShapeDtypeStruct((NUM_CLASS, BATCH), jnp.float32),
        scratch_types=[
            pltpu.VMEM((IDX_PER_GROUP,), jnp.int32),
            pltpu.VMEM((IDX_PER_GROUP,), jnp.int32),
            pltpu.VMEM((IDX_PER_GROUP, EMBED_DIM), jnp.float32),
            pltpu.VMEM((IDX_PER_GROUP, EMBED_DIM), jnp.float32),
            pltpu.VMEM((GROUP, EMBED_DIM), jnp.float32),
            pltpu.VMEM((NUM_CLASS, GROUP), jnp.float32),
            pltpu.VMEM((NUM_CLASS, GROUP), jnp.float32),
            pltpu.VMEM((4, 16), jnp.float32),
            pltpu.SemaphoreType.DMA,
            pltpu.SemaphoreType.DMA,
            pltpu.SemaphoreType.DMA,
            pltpu.SemaphoreType.DMA,
            pltpu.SemaphoreType.DMA,
        ],
        compiler_params=pltpu.CompilerParams(needs_layout_passes=False,
                                             use_tc_tiling_on_sc=False),
    )(_sc_body)
    return kern(text_flat, table_lin, wb)


def kernel(text, table, W, b):
    tt = jnp.transpose(table)
    tail = lax.slice(tt, (0, TAIL_START), (EMBED_DIM, VOCAB))
    tail = jnp.pad(tail, ((0, 0), (0, SUP - (VOCAB - TAIL_START))))
    x = _detranspose(tt, tail)
    table_lin = x.reshape(XROWS * 8, EMBED_DIM)
    wt = jnp.transpose(W) * (1.0 / HIST)
    bb = jnp.broadcast_to(b[:, None], (NUM_CLASS, 16))
    wb = jnp.concatenate([wt, bb], axis=0).astype(jnp.float32)
    out = _classify(text.reshape(-1), table_lin, wb)
    return jnp.transpose(out)
